# baseline (speedup 1.0000x reference)
import jax
import jax.numpy as jnp
from jax.experimental import pallas as pl

_SEQ_BLOCK = 1024


def _emb_kernel(inp_ref, w_ref, out_ref):
    b = pl.program_id(1)
    m = (inp_ref[b, :] != 0).astype(w_ref.dtype)
    w = w_ref[...]
    out_ref[...] = (w * m[:, None])[None]


def kernel(input_tensor, weights):
    batch, seq_len = input_tensor.shape
    dim = weights.shape[1]
    w_main = jax.lax.slice(weights, (1, 0), (1 + seq_len, dim))

    s_blk = _SEQ_BLOCK if seq_len % _SEQ_BLOCK == 0 else seq_len
    grid = (seq_len // s_blk, batch)
    out = pl.pallas_call(
        _emb_kernel,
        grid=grid,
        in_specs=[
            pl.BlockSpec((batch, s_blk), lambda i, b: (0, i)),
            pl.BlockSpec((s_blk, dim), lambda i, b: (i, 0)),
        ],
        out_specs=pl.BlockSpec((1, s_blk, dim), lambda i, b: (b, i, 0)),
        out_shape=jax.ShapeDtypeStruct((batch, seq_len, dim), weights.dtype),
    )(input_tensor, w_main)
    return out


# speedup vs baseline: 1.1644x; 1.1644x over previous
import jax
import jax.numpy as jnp
from jax.experimental import pallas as pl
from jax.experimental.pallas import tpu as pltpu

_SEQ_BLOCK = 1024


def _emb_kernel(inp_ref, w_ref, out_ref):
    m = (inp_ref[...] != 0).astype(w_ref.dtype)
    w = w_ref[...]
    out_ref[...] = w[None, :, :] * m[:, :, None]


def kernel(input_tensor, weights):
    batch, seq_len = input_tensor.shape
    dim = weights.shape[1]
    w_main = jax.lax.slice(weights, (1, 0), (1 + seq_len, dim))

    s_blk = _SEQ_BLOCK if seq_len % _SEQ_BLOCK == 0 else seq_len
    grid = (seq_len // s_blk,)
    out = pl.pallas_call(
        _emb_kernel,
        grid=grid,
        in_specs=[
            pl.BlockSpec((batch, s_blk), lambda i: (0, i)),
            pl.BlockSpec((s_blk, dim), lambda i: (i, 0)),
        ],
        out_specs=pl.BlockSpec((batch, s_blk, dim), lambda i: (0, i, 0)),
        out_shape=jax.ShapeDtypeStruct((batch, seq_len, dim), weights.dtype),
        compiler_params=pltpu.CompilerParams(
            dimension_semantics=("parallel",),
        ),
    )(input_tensor, w_main)
    return out
